# Initial kernel scaffold; baseline (speedup 1.0000x reference)
import jax
import jax.numpy as jnp
from jax.experimental import pallas as pl


def kernel(batch_base_seq, batch_ref_seq, read_table, ref_table):
    raise NotImplementedError("write your pallas kernel here")


# speedup vs baseline: 7.0811x; 7.0811x over previous
import functools

import jax
import jax.numpy as jnp
from jax.experimental import pallas as pl
from jax.experimental.pallas import tpu as pltpu

M = 2048


def _embed_body(base_ref, refi_ref, tab_ref, out_ref):
    cidx = base_ref[...] * 5 + refi_ref[...]
    iota = jax.lax.broadcasted_iota(jnp.int32, (M, 32), 1)
    onehot = (cidx == iota).astype(jnp.float32)
    out_ref[...] = jax.lax.dot_general(
        onehot, tab_ref[...],
        dimension_numbers=(((1,), (0,)), ((), ())),
        preferred_element_type=jnp.float32,
    )


@jax.jit
def kernel(batch_base_seq, batch_ref_seq, read_table, ref_table):
    B, S = batch_base_seq.shape
    D = read_table.shape[1]
    N = B * S
    c = jnp.arange(20)
    combined = jnp.concatenate(
        [read_table[c // 5], ref_table[c % 5]], axis=1)
    tab = jnp.pad(combined, ((0, 12), (0, 0)))
    base = batch_base_seq.astype(jnp.int32).reshape(N, 1)
    refi = batch_ref_seq.astype(jnp.int32).reshape(N, 1)

    out = pl.pallas_call(
        _embed_body,
        grid=(N // M,),
        in_specs=[
            pl.BlockSpec((M, 1), lambda i: (i, 0)),
            pl.BlockSpec((M, 1), lambda i: (i, 0)),
            pl.BlockSpec((32, 2 * D), lambda i: (0, 0)),
        ],
        out_specs=pl.BlockSpec((M, 2 * D), lambda i: (i, 0)),
        out_shape=jax.ShapeDtypeStruct((N, 2 * D), jnp.float32),
    )(base, refi, tab)
    return out.reshape(B, S, 2 * D)
